# Initial kernel scaffold; baseline (speedup 1.0000x reference)
import jax
import jax.numpy as jnp
from jax.experimental import pallas as pl


def kernel(x, table):
    raise NotImplementedError("write your pallas kernel here")


# speedup vs baseline: 2.4907x; 2.4907x over previous
import functools

import jax
import jax.numpy as jnp
from jax import lax
from jax.experimental import pallas as pl
from jax.experimental.pallas import tpu as pltpu
from jax.experimental.pallas import tpu_sc as plsc

NUM_EMB = 1_000_000
DIM = 16
NC = 2
NS = 16
NW = NC * NS
CHUNK = 2048


@functools.partial(jax.jit, static_argnames=("total",))
def _gather_flat(idx, table, total):
    b_per_w = total // NW
    n_chunks = b_per_w // CHUNK
    mesh = plsc.VectorSubcoreMesh(core_axis_name="c", subcore_axis_name="s")

    @functools.partial(
        pl.kernel,
        mesh=mesh,
        compiler_params=pltpu.CompilerParams(use_tc_tiling_on_sc=False),
        out_type=jax.ShapeDtypeStruct((total, DIM), jnp.float32),
        scratch_types=[
            pltpu.VMEM((CHUNK,), jnp.int32),
            pltpu.VMEM((CHUNK, DIM), jnp.float32),
            pltpu.SemaphoreType.DMA,
        ],
    )
    def emb(idx_hbm, table_hbm, out_hbm, idx_v, rows_v, sem):
        wid = lax.axis_index("s") * NC + lax.axis_index("c")
        base0 = wid * b_per_w

        def body(g, carry):
            base = base0 + g * CHUNK
            pltpu.sync_copy(idx_hbm.at[pl.ds(base, CHUNK)], idx_v)
            pltpu.async_copy(table_hbm.at[idx_v], rows_v, sem).wait()
            pltpu.sync_copy(rows_v, out_hbm.at[pl.ds(base, CHUNK)])
            return carry

        lax.fori_loop(0, n_chunks, body, 0)

    return emb(idx, table)


def kernel(x, table):
    idx = x.reshape(-1).astype(jnp.int32)
    out = _gather_flat(idx, table, idx.shape[0])
    return out.reshape(x.shape + (DIM,))


# speedup vs baseline: 2.5676x; 1.0309x over previous
import functools

import jax
import jax.numpy as jnp
from jax import lax
from jax.experimental import pallas as pl
from jax.experimental.pallas import tpu as pltpu
from jax.experimental.pallas import tpu_sc as plsc

NUM_EMB = 1_000_000
DIM = 16
NC = 2
NS = 16
NW = NC * NS
CHUNK = 3200


@functools.partial(jax.jit, static_argnames=("total",))
def _gather_flat(idx, table, total):
    b_per_w = total // NW
    n_chunks = b_per_w // CHUNK
    assert n_chunks % 2 == 0 and n_chunks * CHUNK == b_per_w
    mesh = plsc.VectorSubcoreMesh(core_axis_name="c", subcore_axis_name="s")

    @functools.partial(
        pl.kernel,
        mesh=mesh,
        compiler_params=pltpu.CompilerParams(use_tc_tiling_on_sc=False),
        out_type=jax.ShapeDtypeStruct((total, DIM), jnp.float32),
        scratch_types=[
            pltpu.VMEM((CHUNK,), jnp.int32),
            pltpu.VMEM((CHUNK,), jnp.int32),
            pltpu.VMEM((CHUNK, DIM), jnp.float32),
            pltpu.VMEM((CHUNK, DIM), jnp.float32),
            pltpu.SemaphoreType.DMA,
            pltpu.SemaphoreType.DMA,
            pltpu.SemaphoreType.DMA,
            pltpu.SemaphoreType.DMA,
            pltpu.SemaphoreType.DMA,
            pltpu.SemaphoreType.DMA,
        ],
    )
    def emb(idx_hbm, table_hbm, out_hbm, idx0, idx1, rows0, rows1,
            si0, si1, sg0, sg1, so0, so1):
        wid = lax.axis_index("s") * NC + lax.axis_index("c")
        base0 = wid * b_per_w
        idx_v = (idx0, idx1)
        rows_v = (rows0, rows1)
        s_i = (si0, si1)
        s_g = (sg0, sg1)
        s_o = (so0, so1)

        def idx_cp(g, b):
            return pltpu.make_async_copy(
                idx_hbm.at[pl.ds(base0 + g * CHUNK, CHUNK)], idx_v[b], s_i[b])

        def gat_cp(b):
            return pltpu.make_async_copy(table_hbm.at[idx_v[b]], rows_v[b], s_g[b])

        def out_cp(g, b):
            return pltpu.make_async_copy(
                rows_v[b], out_hbm.at[pl.ds(base0 + g * CHUNK, CHUNK)], s_o[b])

        idx_cp(0, 0).start()
        idx_cp(1, 1).start()
        idx_cp(0, 0).wait()
        gat_cp(0).start()

        def body(k, carry):
            g = 2 * k
            @pl.when(k > 0)
            def _():
                out_cp(g - 1, 1).wait()
            idx_cp(g + 1, 1).wait()
            gat_cp(1).start()
            gat_cp(0).wait()
            out_cp(g, 0).start()
            @pl.when(g + 2 < n_chunks)
            def _():
                idx_cp(g + 2, 0).start()
            out_cp(g, 0).wait()
            @pl.when(g + 2 < n_chunks)
            def _():
                idx_cp(g + 2, 0).wait()
                gat_cp(0).start()
            gat_cp(1).wait()
            out_cp(g + 1, 1).start()
            @pl.when(g + 3 < n_chunks)
            def _():
                idx_cp(g + 3, 1).start()
            return carry

        lax.fori_loop(0, n_chunks // 2, body, 0)
        out_cp(n_chunks - 1, 1).wait()

    return emb(idx, table)


def kernel(x, table):
    idx = x.reshape(-1).astype(jnp.int32)
    out = _gather_flat(idx, table, idx.shape[0])
    return out.reshape(x.shape + (DIM,))
